# baseline (speedup 1.0000x reference)
import functools

import numpy as np
import jax
import jax.numpy as jnp
from jax import lax
from jax.experimental import pallas as pl
from jax.experimental.pallas import tpu as pltpu
from jax.experimental.pallas import tpu_sc as plsc

N_NODES = 100000
DMAX = 32
S = 16
B = 4096
GD = 128
NW = 32
B_PER_W = B // NW

ROWS = 2048
GRID = (N_NODES + ROWS - 1) // ROWS
PAD = GRID * ROWS
LROWS = PAD // 128

AROWS = B * DMAX // 128
CROWS = B * S // 128


@functools.cache
def _sc_target_fn():
    mesh = plsc.VectorSubcoreMesh(core_axis_name="c", subcore_axis_name="s")

    @functools.partial(
        pl.kernel,
        mesh=mesh,
        out_type=jax.ShapeDtypeStruct((8, GD), jnp.float32),
        scratch_types=[
            pltpu.VMEM((8,), jnp.int32),
            pltpu.VMEM((8, GD), jnp.float32),
            pltpu.SemaphoreType.DMA,
        ],
    )
    def body(ids_hbm, elms_hbm, t_out, idx8_v, t_v, sem):
        wid = lax.axis_index("s") * 2 + lax.axis_index("c")

        @pl.when(wid == 0)
        def _():
            pltpu.sync_copy(ids_hbm.at[pl.ds(0, 8)], idx8_v)
            pltpu.async_copy(elms_hbm.at[idx8_v], t_v, sem).wait()
            pltpu.sync_copy(t_v, t_out)

    return body


@functools.cache
def _sc_gather_adj_fn():
    mesh = plsc.VectorSubcoreMesh(core_axis_name="c", subcore_axis_name="s")

    @functools.partial(
        pl.kernel,
        mesh=mesh,
        compiler_params=pltpu.CompilerParams(use_tc_tiling_on_sc=False),
        out_type=(
            jax.ShapeDtypeStruct((AROWS, 128), jnp.int32),
            jax.ShapeDtypeStruct((AROWS, 128), jnp.int32),
        ),
        scratch_types=[
            pltpu.VMEM((B_PER_W,), jnp.int32),
            pltpu.VMEM((B_PER_W, DMAX), jnp.int32),
            pltpu.VMEM((DMAX, 128), jnp.int32),
            pltpu.SemaphoreType.DMA,
        ],
    )
    def body(ids_hbm, gids_hbm, adj_hbm, gadj_hbm,
             a_out, g_out, idx_v, rows_v, buf_v, sem):
        wid = lax.axis_index("s") * 2 + lax.axis_index("c")
        base = wid * B_PER_W

        def one_table(src_ids_hbm, table_hbm, out_hbm):
            pltpu.sync_copy(src_ids_hbm.at[pl.ds(base, B_PER_W)], idx_v)
            pltpu.async_copy(table_hbm.at[idx_v], rows_v, sem).wait()
            for b in range(B_PER_W):
                r, cc = b >> 2, (b & 3) * DMAX
                buf_v[r, pl.ds(cc, 16)] = rows_v[b, pl.ds(0, 16)]
                buf_v[r, pl.ds(cc + 16, 16)] = rows_v[b, pl.ds(16, 16)]
            pltpu.sync_copy(buf_v, out_hbm.at[pl.ds(wid * DMAX, DMAX)])

        one_table(ids_hbm, adj_hbm, a_out)
        one_table(gids_hbm, gadj_hbm, g_out)

    return body


def _logit_body(t_ref, e_ref, o_ref):
    e = e_ref[...]
    t = t_ref[0:1, :]
    d = t - e
    d2 = jnp.sum(d * d, axis=1)
    dist = jnp.sqrt(d2)
    w = jnp.ones_like(dist) / dist
    o_ref[...] = jnp.log(w).reshape(ROWS // 128, 128)


def _tc_logits(t8, geto_elms):
    return pl.pallas_call(
        _logit_body,
        grid=(GRID,),
        in_specs=[
            pl.BlockSpec((8, GD), lambda i: (0, 0)),
            pl.BlockSpec((ROWS, GD), lambda i: (i, 0)),
        ],
        out_specs=pl.BlockSpec((ROWS // 128, 128), lambda i: (i, 0)),
        out_shape=jax.ShapeDtypeStruct((LROWS, 128), jnp.float32),
    )(t8, geto_elms)


@functools.cache
def _sc_gather_logits_fn():
    mesh = plsc.VectorSubcoreMesh(core_axis_name="c", subcore_axis_name="s")

    @functools.partial(
        pl.kernel,
        mesh=mesh,
        compiler_params=pltpu.CompilerParams(needs_layout_passes=False),
        out_type=jax.ShapeDtypeStruct((CROWS, 128), jnp.float32),
        scratch_types=[
            pltpu.VMEM((LROWS, 128), jnp.float32),
            pltpu.VMEM((DMAX, 128), jnp.int32),
            pltpu.VMEM((S, 128), jnp.float32),
        ],
    )
    def body(g4_hbm, logit_hbm, out_hbm, table_v, g_v, out_v):
        wid = lax.axis_index("s") * 2 + lax.axis_index("c")
        pltpu.sync_copy(logit_hbm, table_v)
        pltpu.sync_copy(g4_hbm.at[pl.ds(wid * DMAX, DMAX)], g_v)
        for b in range(B_PER_W):
            idx16 = g_v[b >> 2, pl.ds((b & 3) * DMAX, 16)]
            row = lax.shift_right_logical(idx16, 7)
            col = idx16 & 127
            val = plsc.load_gather(table_v, [row, col])
            out_v[b >> 3, pl.ds((b & 7) * 16, 16)] = val
        pltpu.sync_copy(out_v, out_hbm.at[pl.ds(wid * S, S)])

    return body


def _sample_body(l2_ref, a_ref, g_ref, gum_ref, o1_ref, o2_ref):
    lg = l2_ref[...]
    j2d = (128 * lax.broadcasted_iota(jnp.int32, (CROWS, 128), 0)
           + lax.broadcasted_iota(jnp.int32, (CROWS, 128), 1))
    io_c = lax.broadcasted_iota(jnp.int32, (128, 64), 0)
    io_col = lax.broadcasted_iota(jnp.int32, (128, 64), 1)
    mm = lax.shift_right_logical(io_col, 4) * DMAX
    kk = io_col & 15
    io_fc = lax.broadcasted_iota(jnp.int32, (AROWS, 64), 1) & 15
    oh = jnp.zeros((128, 64), jnp.float32)
    fill = jnp.zeros((AROWS, 64), jnp.int32)
    for k in range(S):
        val = lg + gum_ref[k]
        m = jnp.max(val)
        cand = jnp.where(val == m, j2d, jnp.int32(2 ** 30))
        wk = jnp.min(cand)
        valid = wk < DMAX
        oh = oh + jnp.where((kk == k) & (io_c == mm + wk) & valid, 1.0, 0.0)
        fill = fill + jnp.where((io_fc == k) & (~valid), 1, 0)
    dn = (((1,), (0,)), ((), ()))
    int_min = jnp.int32(-(2 ** 31))
    o1 = lax.dot_general(a_ref[...].astype(jnp.float32), oh, dn,
                         preferred_element_type=jnp.float32)
    o2 = lax.dot_general(g_ref[...].astype(jnp.float32), oh, dn,
                         preferred_element_type=jnp.float32)
    o1_ref[...] = jnp.where(fill > 0, int_min, o1.astype(jnp.int32))
    o2_ref[...] = jnp.where(fill > 0, int_min, o2.astype(jnp.int32))


def _tc_sample(l2, a4, g4, gum):
    return pl.pallas_call(
        _sample_body,
        out_shape=(
            jax.ShapeDtypeStruct((AROWS, 64), jnp.int32),
            jax.ShapeDtypeStruct((AROWS, 64), jnp.int32),
        ),
    )(l2, a4, g4, gum)


_GUM = None


def _np_threefry2x32_bits(seed, n):
    def rotl(v, r):
        return ((v << np.uint32(r)) | (v >> np.uint32(32 - r))).astype(
            np.uint32)

    k0 = np.uint32(seed >> 32)
    k1 = np.uint32(seed & 0xFFFFFFFF)
    ks2 = np.uint32(k0 ^ k1 ^ np.uint32(0x1BD11BDA))
    rot = ((13, 15, 26, 6), (17, 29, 16, 24))
    sched = ((k1, ks2), (ks2, k0), (k0, k1), (k1, ks2), (ks2, k0))
    x0 = np.full(n, k0, dtype=np.uint32)
    x1 = (np.arange(n, dtype=np.uint32) + k1).astype(np.uint32)
    for i in range(5):
        for r in rot[i % 2]:
            x0 = (x0 + x1).astype(np.uint32)
            x1 = rotl(x1, r)
            x1 = x1 ^ x0
        x0 = (x0 + sched[i][0]).astype(np.uint32)
        x1 = (x1 + sched[i][1] + np.uint32(i + 1)).astype(np.uint32)
    return x0 ^ x1


def _gumbel_const():
    global _GUM
    if _GUM is None:
        bits = _np_threefry2x32_bits(42, DMAX * B * S)[: S * B * S]
        float_bits = (bits >> np.uint32(9)) | np.uint32(0x3F800000)
        f = float_bits.view(np.float32) - np.float32(1.0)
        tiny = np.float32(np.finfo(np.float32).tiny)
        u = np.maximum(tiny, (f * (np.float32(1.0) - tiny) + tiny).astype(
            np.float32))
        g = (-np.log(-np.log(u))).astype(np.float32)
        _GUM = g.reshape(S, CROWS, 128)
    return _GUM


def kernel(ids, geto_ids, geto_elms, adj_info, geto_adj_info, num_samples,
           support_size, batch_size, geto_dims, hop, resampling_rate):
    t8 = _sc_target_fn()(ids, geto_elms)
    logit_all = _tc_logits(t8, geto_elms)
    a4, g4 = _sc_gather_adj_fn()(ids, geto_ids, adj_info, geto_adj_info)
    l2 = _sc_gather_logits_fn()(g4, logit_all)
    gum = jnp.asarray(_gumbel_const())
    o1, o2 = _tc_sample(l2, a4, g4, gum)
    return o1.reshape(B, S), o2.reshape(B, S)


# speedup vs baseline: 1.3217x; 1.3217x over previous
import functools

import numpy as np
import jax
import jax.numpy as jnp
from jax import lax
from jax.experimental import pallas as pl
from jax.experimental.pallas import tpu as pltpu
from jax.experimental.pallas import tpu_sc as plsc

N_NODES = 100000
DMAX = 32
S = 16
B = 4096
GD = 128
NW = 32
B_PER_W = B // NW

ROWS = 2048
GRID = (N_NODES + ROWS - 1) // ROWS
PAD = GRID * ROWS
LROWS = PAD // 128

AROWS = B * DMAX // 128
CROWS = B * S // 128


@functools.cache
def _sc_target_fn():
    mesh = plsc.VectorSubcoreMesh(core_axis_name="c", subcore_axis_name="s")

    @functools.partial(
        pl.kernel,
        mesh=mesh,
        out_type=jax.ShapeDtypeStruct((8, GD), jnp.float32),
        scratch_types=[
            pltpu.VMEM((8,), jnp.int32),
            pltpu.VMEM((8, GD), jnp.float32),
            pltpu.SemaphoreType.DMA,
        ],
    )
    def body(ids_hbm, elms_hbm, t_out, idx8_v, t_v, sem):
        wid = lax.axis_index("s") * 2 + lax.axis_index("c")

        @pl.when(wid == 0)
        def _():
            pltpu.sync_copy(ids_hbm.at[pl.ds(0, 8)], idx8_v)
            pltpu.async_copy(elms_hbm.at[idx8_v], t_v, sem).wait()
            pltpu.sync_copy(t_v, t_out)

    return body


@functools.cache
def _sc_gather_one_fn():
    mesh = plsc.VectorSubcoreMesh(core_axis_name="c", subcore_axis_name="s")

    @functools.partial(
        pl.kernel,
        mesh=mesh,
        compiler_params=pltpu.CompilerParams(use_tc_tiling_on_sc=False),
        out_type=jax.ShapeDtypeStruct((AROWS, 128), jnp.int32),
        scratch_types=[
            pltpu.VMEM((B_PER_W,), jnp.int32),
            pltpu.VMEM((B_PER_W, DMAX), jnp.int32),
            pltpu.VMEM((DMAX, 128), jnp.int32),
            pltpu.SemaphoreType.DMA,
        ],
    )
    def body(src_ids_hbm, table_hbm, out_hbm, idx_v, rows_v, buf_v, sem):
        wid = lax.axis_index("s") * 2 + lax.axis_index("c")
        base = wid * B_PER_W
        pltpu.sync_copy(src_ids_hbm.at[pl.ds(base, B_PER_W)], idx_v)
        pltpu.async_copy(table_hbm.at[idx_v], rows_v, sem).wait()
        for b in range(B_PER_W):
            r, cc = b >> 2, (b & 3) * DMAX
            buf_v[r, pl.ds(cc, 16)] = rows_v[b, pl.ds(0, 16)]
            buf_v[r, pl.ds(cc + 16, 16)] = rows_v[b, pl.ds(16, 16)]
        pltpu.sync_copy(buf_v, out_hbm.at[pl.ds(wid * DMAX, DMAX)])

    return body


def _logit_body(t_ref, e_ref, o_ref):
    e = e_ref[...]
    t = t_ref[0:1, :]
    d = t - e
    d2 = jnp.sum(d * d, axis=1)
    dist = jnp.sqrt(d2)
    w = jnp.ones_like(dist) / dist
    o_ref[...] = jnp.log(w).reshape(ROWS // 128, 128)


def _tc_logits(t8, geto_elms):
    return pl.pallas_call(
        _logit_body,
        grid=(GRID,),
        in_specs=[
            pl.BlockSpec((8, GD), lambda i: (0, 0)),
            pl.BlockSpec((ROWS, GD), lambda i: (i, 0)),
        ],
        out_specs=pl.BlockSpec((ROWS // 128, 128), lambda i: (i, 0)),
        out_shape=jax.ShapeDtypeStruct((LROWS, 128), jnp.float32),
    )(t8, geto_elms)


@functools.cache
def _sc_gather_logits_fn():
    mesh = plsc.VectorSubcoreMesh(core_axis_name="c", subcore_axis_name="s")

    @functools.partial(
        pl.kernel,
        mesh=mesh,
        compiler_params=pltpu.CompilerParams(needs_layout_passes=False),
        out_type=jax.ShapeDtypeStruct((CROWS, 128), jnp.float32),
        scratch_types=[
            pltpu.VMEM((LROWS, 128), jnp.float32),
            pltpu.VMEM((DMAX, 128), jnp.int32),
            pltpu.VMEM((S, 128), jnp.float32),
        ],
    )
    def body(g4_hbm, logit_hbm, out_hbm, table_v, g_v, out_v):
        wid = lax.axis_index("s") * 2 + lax.axis_index("c")
        pltpu.sync_copy(logit_hbm, table_v)
        pltpu.sync_copy(g4_hbm.at[pl.ds(wid * DMAX, DMAX)], g_v)
        for b in range(B_PER_W):
            idx16 = g_v[b >> 2, pl.ds((b & 3) * DMAX, 16)]
            row = lax.shift_right_logical(idx16, 7)
            col = idx16 & 127
            val = plsc.load_gather(table_v, [row, col])
            out_v[b >> 3, pl.ds((b & 7) * 16, 16)] = val
        pltpu.sync_copy(out_v, out_hbm.at[pl.ds(wid * S, S)])

    return body


def _argmax_body(l2_ref, gum_ref, w_ref):
    lg = l2_ref[...]
    j2d = (128 * lax.broadcasted_iota(jnp.int32, (CROWS, 128), 0)
           + lax.broadcasted_iota(jnp.int32, (CROWS, 128), 1))
    io_c = lax.broadcasted_iota(jnp.int32, (8, 128), 1)
    wout = jnp.zeros((8, 128), jnp.int32)
    for k in range(S):
        val = lg + gum_ref[k]
        m = jnp.max(val)
        cand = jnp.where(val == m, j2d, jnp.int32(2 ** 30))
        wk = jnp.min(cand)
        wout = wout + jnp.where(io_c == k, wk, 0)
    w_ref[...] = wout


def _tc_argmax(l2, gum):
    return pl.pallas_call(
        _argmax_body,
        out_shape=jax.ShapeDtypeStruct((8, 128), jnp.int32),
    )(l2, gum)


def _select_body(w_ref, a_ref, g_ref, o1_ref, o2_ref):
    wrow = w_ref[...][0:1, :]
    io_c = lax.broadcasted_iota(jnp.int32, (128, 64), 0)
    io_col = lax.broadcasted_iota(jnp.int32, (128, 64), 1)
    mm = lax.shift_right_logical(io_col, 4) * DMAX
    kk = io_col & 15
    io_fc = lax.broadcasted_iota(jnp.int32, (AROWS, 64), 1) & 15
    oh = jnp.zeros((128, 64), jnp.float32)
    fill = jnp.zeros((AROWS, 64), jnp.int32)
    for k in range(S):
        wk = wrow[0, k]
        valid = wk < DMAX
        oh = oh + jnp.where((kk == k) & (io_c == mm + wk) & valid, 1.0, 0.0)
        fill = fill + jnp.where((io_fc == k) & (~valid), 1, 0)
    dn = (((1,), (0,)), ((), ()))
    int_min = jnp.int32(-(2 ** 31))
    o1 = lax.dot_general(a_ref[...].astype(jnp.float32), oh, dn,
                         preferred_element_type=jnp.float32)
    o2 = lax.dot_general(g_ref[...].astype(jnp.float32), oh, dn,
                         preferred_element_type=jnp.float32)
    o1_ref[...] = jnp.where(fill > 0, int_min, o1.astype(jnp.int32))
    o2_ref[...] = jnp.where(fill > 0, int_min, o2.astype(jnp.int32))


def _tc_select(w8, a4, g4):
    return pl.pallas_call(
        _select_body,
        out_shape=(
            jax.ShapeDtypeStruct((AROWS, 64), jnp.int32),
            jax.ShapeDtypeStruct((AROWS, 64), jnp.int32),
        ),
    )(w8, a4, g4)


_GUM = None


def _np_threefry2x32_bits(seed, n):
    def rotl(v, r):
        return ((v << np.uint32(r)) | (v >> np.uint32(32 - r))).astype(
            np.uint32)

    k0 = np.uint32(seed >> 32)
    k1 = np.uint32(seed & 0xFFFFFFFF)
    ks2 = np.uint32(k0 ^ k1 ^ np.uint32(0x1BD11BDA))
    rot = ((13, 15, 26, 6), (17, 29, 16, 24))
    sched = ((k1, ks2), (ks2, k0), (k0, k1), (k1, ks2), (ks2, k0))
    x0 = np.full(n, k0, dtype=np.uint32)
    x1 = (np.arange(n, dtype=np.uint32) + k1).astype(np.uint32)
    for i in range(5):
        for r in rot[i % 2]:
            x0 = (x0 + x1).astype(np.uint32)
            x1 = rotl(x1, r)
            x1 = x1 ^ x0
        x0 = (x0 + sched[i][0]).astype(np.uint32)
        x1 = (x1 + sched[i][1] + np.uint32(i + 1)).astype(np.uint32)
    return x0 ^ x1


def _gumbel_const():
    global _GUM
    if _GUM is None:
        bits = _np_threefry2x32_bits(42, DMAX * B * S)[: S * B * S]
        float_bits = (bits >> np.uint32(9)) | np.uint32(0x3F800000)
        f = float_bits.view(np.float32) - np.float32(1.0)
        tiny = np.float32(np.finfo(np.float32).tiny)
        u = np.maximum(tiny, (f * (np.float32(1.0) - tiny) + tiny).astype(
            np.float32))
        g = (-np.log(-np.log(u))).astype(np.float32)
        _GUM = g.reshape(S, CROWS, 128)
    return _GUM


def kernel(ids, geto_ids, geto_elms, adj_info, geto_adj_info, num_samples,
           support_size, batch_size, geto_dims, hop, resampling_rate):
    t8 = _sc_target_fn()(ids, geto_elms)
    logit_all = _tc_logits(t8, geto_elms)
    g4 = _sc_gather_one_fn()(geto_ids, geto_adj_info)
    l2 = _sc_gather_logits_fn()(g4, logit_all)
    gum = jnp.asarray(_gumbel_const())
    w8 = _tc_argmax(l2, gum)

    def _with_gather(operands):
        ids_, adj_, g4_, w8_ = operands
        a4 = _sc_gather_one_fn()(ids_, adj_)
        o1, o2 = _tc_select(w8_, a4, g4_)
        return o1.reshape(B, S), o2.reshape(B, S)

    def _all_fill(operands):
        del operands
        full = jnp.full((B, S), jnp.int32(-(2 ** 31)), jnp.int32)
        return full, full

    any_valid = jnp.any(w8[0, :S] < DMAX)
    return lax.cond(any_valid, _with_gather, _all_fill,
                    (ids, adj_info, g4, w8))


# speedup vs baseline: 1.4433x; 1.0919x over previous
import functools

import numpy as np
import jax
import jax.numpy as jnp
from jax import lax
from jax.experimental import pallas as pl
from jax.experimental.pallas import tpu as pltpu
from jax.experimental.pallas import tpu_sc as plsc

N_NODES = 100000
DMAX = 32
S = 16
B = 4096
GD = 128
NW = 32
B_PER_W = B // NW

ROWS = 4096
GRID = (N_NODES + ROWS - 1) // ROWS
PAD = GRID * ROWS
LROWS = PAD // 128

AROWS = B * DMAX // 128
CROWS = B * S // 128


@functools.cache
def _sc_target_fn():
    mesh = plsc.VectorSubcoreMesh(core_axis_name="c", subcore_axis_name="s")

    @functools.partial(
        pl.kernel,
        mesh=mesh,
        out_type=jax.ShapeDtypeStruct((8, GD), jnp.float32),
        scratch_types=[
            pltpu.VMEM((8,), jnp.int32),
            pltpu.VMEM((8, GD), jnp.float32),
            pltpu.SemaphoreType.DMA,
        ],
    )
    def body(ids_hbm, elms_hbm, t_out, idx8_v, t_v, sem):
        wid = lax.axis_index("s") * 2 + lax.axis_index("c")

        @pl.when(wid == 0)
        def _():
            pltpu.sync_copy(ids_hbm.at[pl.ds(0, 8)], idx8_v)
            pltpu.async_copy(elms_hbm.at[idx8_v], t_v, sem).wait()
            pltpu.sync_copy(t_v, t_out)

    return body


@functools.cache
def _sc_gather_one_fn():
    mesh = plsc.VectorSubcoreMesh(core_axis_name="c", subcore_axis_name="s")

    @functools.partial(
        pl.kernel,
        mesh=mesh,
        compiler_params=pltpu.CompilerParams(use_tc_tiling_on_sc=False),
        out_type=jax.ShapeDtypeStruct((AROWS, 128), jnp.int32),
        scratch_types=[
            pltpu.VMEM((B_PER_W,), jnp.int32),
            pltpu.VMEM((B_PER_W, DMAX), jnp.int32),
            pltpu.VMEM((DMAX, 128), jnp.int32),
            pltpu.SemaphoreType.DMA,
        ],
    )
    def body(src_ids_hbm, table_hbm, out_hbm, idx_v, rows_v, buf_v, sem):
        wid = lax.axis_index("s") * 2 + lax.axis_index("c")
        base = wid * B_PER_W
        pltpu.sync_copy(src_ids_hbm.at[pl.ds(base, B_PER_W)], idx_v)
        pltpu.async_copy(table_hbm.at[idx_v], rows_v, sem).wait()
        for b in range(B_PER_W):
            r, cc = b >> 2, (b & 3) * DMAX
            buf_v[r, pl.ds(cc, 16)] = rows_v[b, pl.ds(0, 16)]
            buf_v[r, pl.ds(cc + 16, 16)] = rows_v[b, pl.ds(16, 16)]
        pltpu.sync_copy(buf_v, out_hbm.at[pl.ds(wid * DMAX, DMAX)])

    return body


def _logit_body(t_ref, e_ref, o_ref):
    e = e_ref[...]
    t = t_ref[0:1, :]
    d = t - e
    d2 = jnp.sum(d * d, axis=1)
    dist = jnp.sqrt(d2)
    w = jnp.ones_like(dist) / dist
    o_ref[...] = jnp.log(w).reshape(ROWS // 128, 128)


def _tc_logits(t8, geto_elms):
    return pl.pallas_call(
        _logit_body,
        grid=(GRID,),
        in_specs=[
            pl.BlockSpec((8, GD), lambda i: (0, 0)),
            pl.BlockSpec((ROWS, GD), lambda i: (i, 0)),
        ],
        out_specs=pl.BlockSpec((ROWS // 128, 128), lambda i: (i, 0)),
        out_shape=jax.ShapeDtypeStruct((LROWS, 128), jnp.float32),
    )(t8, geto_elms)


@functools.cache
def _sc_gather_logits_fn():
    mesh = plsc.VectorSubcoreMesh(core_axis_name="c", subcore_axis_name="s")

    @functools.partial(
        pl.kernel,
        mesh=mesh,
        compiler_params=pltpu.CompilerParams(needs_layout_passes=False),
        out_type=jax.ShapeDtypeStruct((CROWS, 128), jnp.float32),
        scratch_types=[
            pltpu.VMEM((LROWS, 128), jnp.float32),
            pltpu.VMEM((DMAX, 128), jnp.int32),
            pltpu.VMEM((S, 128), jnp.float32),
        ],
    )
    def body(g4_hbm, logit_hbm, out_hbm, table_v, g_v, out_v):
        wid = lax.axis_index("s") * 2 + lax.axis_index("c")
        pltpu.sync_copy(logit_hbm, table_v)
        pltpu.sync_copy(g4_hbm.at[pl.ds(wid * DMAX, DMAX)], g_v)
        for b in range(B_PER_W):
            idx16 = g_v[b >> 2, pl.ds((b & 3) * DMAX, 16)]
            row = lax.shift_right_logical(idx16, 7)
            col = idx16 & 127
            val = plsc.load_gather(table_v, [row, col])
            out_v[b >> 3, pl.ds((b & 7) * 16, 16)] = val
        pltpu.sync_copy(out_v, out_hbm.at[pl.ds(wid * S, S)])

    return body


def _argmax_body(l2_ref, gum_ref, w_ref):
    lg = l2_ref[...]
    j2d = (128 * lax.broadcasted_iota(jnp.int32, (CROWS, 128), 0)
           + lax.broadcasted_iota(jnp.int32, (CROWS, 128), 1))
    io_c = lax.broadcasted_iota(jnp.int32, (8, 128), 1)
    wout = jnp.zeros((8, 128), jnp.int32)
    for k in range(S):
        val = lg + gum_ref[k]
        m = jnp.max(val)
        cand = jnp.where(val == m, j2d, jnp.int32(2 ** 30))
        wk = jnp.min(cand)
        wout = wout + jnp.where(io_c == k, wk, 0)
    w_ref[...] = wout


def _tc_argmax(l2, gum):
    return pl.pallas_call(
        _argmax_body,
        out_shape=jax.ShapeDtypeStruct((8, 128), jnp.int32),
    )(l2, gum)


def _select_body(w_ref, a_ref, g_ref, o1_ref, o2_ref):
    wrow = w_ref[...][0:1, :]
    io_c = lax.broadcasted_iota(jnp.int32, (128, 64), 0)
    io_col = lax.broadcasted_iota(jnp.int32, (128, 64), 1)
    mm = lax.shift_right_logical(io_col, 4) * DMAX
    kk = io_col & 15
    io_fc = lax.broadcasted_iota(jnp.int32, (AROWS, 64), 1) & 15
    oh = jnp.zeros((128, 64), jnp.float32)
    fill = jnp.zeros((AROWS, 64), jnp.int32)
    for k in range(S):
        wk = wrow[0, k]
        valid = wk < DMAX
        oh = oh + jnp.where((kk == k) & (io_c == mm + wk) & valid, 1.0, 0.0)
        fill = fill + jnp.where((io_fc == k) & (~valid), 1, 0)
    dn = (((1,), (0,)), ((), ()))
    int_min = jnp.int32(-(2 ** 31))
    o1 = lax.dot_general(a_ref[...].astype(jnp.float32), oh, dn,
                         preferred_element_type=jnp.float32)
    o2 = lax.dot_general(g_ref[...].astype(jnp.float32), oh, dn,
                         preferred_element_type=jnp.float32)
    o1_ref[...] = jnp.where(fill > 0, int_min, o1.astype(jnp.int32))
    o2_ref[...] = jnp.where(fill > 0, int_min, o2.astype(jnp.int32))


def _tc_select(w8, a4, g4):
    return pl.pallas_call(
        _select_body,
        out_shape=(
            jax.ShapeDtypeStruct((AROWS, 64), jnp.int32),
            jax.ShapeDtypeStruct((AROWS, 64), jnp.int32),
        ),
    )(w8, a4, g4)


_GUM = None


def _np_threefry2x32_bits(seed, n):
    def rotl(v, r):
        return ((v << np.uint32(r)) | (v >> np.uint32(32 - r))).astype(
            np.uint32)

    k0 = np.uint32(seed >> 32)
    k1 = np.uint32(seed & 0xFFFFFFFF)
    ks2 = np.uint32(k0 ^ k1 ^ np.uint32(0x1BD11BDA))
    rot = ((13, 15, 26, 6), (17, 29, 16, 24))
    sched = ((k1, ks2), (ks2, k0), (k0, k1), (k1, ks2), (ks2, k0))
    x0 = np.full(n, k0, dtype=np.uint32)
    x1 = (np.arange(n, dtype=np.uint32) + k1).astype(np.uint32)
    for i in range(5):
        for r in rot[i % 2]:
            x0 = (x0 + x1).astype(np.uint32)
            x1 = rotl(x1, r)
            x1 = x1 ^ x0
        x0 = (x0 + sched[i][0]).astype(np.uint32)
        x1 = (x1 + sched[i][1] + np.uint32(i + 1)).astype(np.uint32)
    return x0 ^ x1


def _gumbel_const():
    global _GUM
    if _GUM is None:
        bits = _np_threefry2x32_bits(42, DMAX * B * S)[: S * B * S]
        float_bits = (bits >> np.uint32(9)) | np.uint32(0x3F800000)
        f = float_bits.view(np.float32) - np.float32(1.0)
        tiny = np.float32(np.finfo(np.float32).tiny)
        u = np.maximum(tiny, (f * (np.float32(1.0) - tiny) + tiny).astype(
            np.float32))
        g = (-np.log(-np.log(u))).astype(np.float32)
        _GUM = g.reshape(S, CROWS, 128)
    return _GUM


def kernel(ids, geto_ids, geto_elms, adj_info, geto_adj_info, num_samples,
           support_size, batch_size, geto_dims, hop, resampling_rate):
    t8 = _sc_target_fn()(ids, geto_elms)
    logit_all = _tc_logits(t8, geto_elms)
    g4 = _sc_gather_one_fn()(geto_ids, geto_adj_info)
    l2 = _sc_gather_logits_fn()(g4, logit_all)
    gum = jnp.asarray(_gumbel_const())
    w8 = _tc_argmax(l2, gum)

    def _with_gather(operands):
        ids_, adj_, g4_, w8_ = operands
        a4 = _sc_gather_one_fn()(ids_, adj_)
        o1, o2 = _tc_select(w8_, a4, g4_)
        return o1.reshape(B, S), o2.reshape(B, S)

    def _all_fill(operands):
        del operands
        full = jnp.full((B, S), jnp.int32(-(2 ** 31)), jnp.int32)
        return full, full

    any_valid = jnp.any(w8[0, :S] < DMAX)
    return lax.cond(any_valid, _with_gather, _all_fill,
                    (ids, adj_info, g4, w8))


# speedup vs baseline: 1.4823x; 1.0271x over previous
import functools

import numpy as np
import jax
import jax.numpy as jnp
from jax import lax
from jax.experimental import pallas as pl
from jax.experimental.pallas import tpu as pltpu
from jax.experimental.pallas import tpu_sc as plsc

N_NODES = 100000
DMAX = 32
S = 16
B = 4096
GD = 128
NW = 32
B_PER_W = B // NW

ROWS = 8192
GRID = (N_NODES + ROWS - 1) // ROWS
PAD = GRID * ROWS
LROWS = PAD // 128

AROWS = B * DMAX // 128
CROWS = B * S // 128


@functools.cache
def _sc_target_fn():
    mesh = plsc.VectorSubcoreMesh(core_axis_name="c", subcore_axis_name="s")

    @functools.partial(
        pl.kernel,
        mesh=mesh,
        out_type=jax.ShapeDtypeStruct((8, GD), jnp.float32),
        scratch_types=[
            pltpu.VMEM((8,), jnp.int32),
            pltpu.VMEM((8, GD), jnp.float32),
            pltpu.SemaphoreType.DMA,
        ],
    )
    def body(ids_hbm, elms_hbm, t_out, idx8_v, t_v, sem):
        wid = lax.axis_index("s") * 2 + lax.axis_index("c")

        @pl.when(wid == 0)
        def _():
            pltpu.sync_copy(ids_hbm.at[pl.ds(0, 8)], idx8_v)
            pltpu.async_copy(elms_hbm.at[idx8_v], t_v, sem).wait()
            pltpu.sync_copy(t_v, t_out)

    return body


@functools.cache
def _sc_gather_one_fn():
    mesh = plsc.VectorSubcoreMesh(core_axis_name="c", subcore_axis_name="s")

    @functools.partial(
        pl.kernel,
        mesh=mesh,
        compiler_params=pltpu.CompilerParams(use_tc_tiling_on_sc=False),
        out_type=jax.ShapeDtypeStruct((AROWS, 128), jnp.int32),
        scratch_types=[
            pltpu.VMEM((B_PER_W,), jnp.int32),
            pltpu.VMEM((B_PER_W, DMAX), jnp.int32),
            pltpu.VMEM((DMAX, 128), jnp.int32),
            pltpu.SemaphoreType.DMA,
        ],
    )
    def body(src_ids_hbm, table_hbm, out_hbm, idx_v, rows_v, buf_v, sem):
        wid = lax.axis_index("s") * 2 + lax.axis_index("c")
        base = wid * B_PER_W
        pltpu.sync_copy(src_ids_hbm.at[pl.ds(base, B_PER_W)], idx_v)
        pltpu.async_copy(table_hbm.at[idx_v], rows_v, sem).wait()
        for b in range(B_PER_W):
            r, cc = b >> 2, (b & 3) * DMAX
            buf_v[r, pl.ds(cc, 16)] = rows_v[b, pl.ds(0, 16)]
            buf_v[r, pl.ds(cc + 16, 16)] = rows_v[b, pl.ds(16, 16)]
        pltpu.sync_copy(buf_v, out_hbm.at[pl.ds(wid * DMAX, DMAX)])

    return body


def _logit_body(t_ref, e_ref, o_ref):
    e = e_ref[...]
    t = t_ref[0:1, :]
    d = t - e
    d2 = jnp.sum(d * d, axis=1)
    dist = jnp.sqrt(d2)
    w = jnp.ones_like(dist) / dist
    o_ref[...] = jnp.log(w).reshape(ROWS // 128, 128)


def _tc_logits(t8, geto_elms):
    return pl.pallas_call(
        _logit_body,
        grid=(GRID,),
        in_specs=[
            pl.BlockSpec((8, GD), lambda i: (0, 0)),
            pl.BlockSpec((ROWS, GD), lambda i: (i, 0)),
        ],
        out_specs=pl.BlockSpec((ROWS // 128, 128), lambda i: (i, 0)),
        out_shape=jax.ShapeDtypeStruct((LROWS, 128), jnp.float32),
    )(t8, geto_elms)


@functools.cache
def _sc_gather_logits_fn():
    mesh = plsc.VectorSubcoreMesh(core_axis_name="c", subcore_axis_name="s")

    @functools.partial(
        pl.kernel,
        mesh=mesh,
        compiler_params=pltpu.CompilerParams(needs_layout_passes=False),
        out_type=jax.ShapeDtypeStruct((CROWS, 128), jnp.float32),
        scratch_types=[
            pltpu.VMEM((LROWS, 128), jnp.float32),
            pltpu.VMEM((DMAX, 128), jnp.int32),
            pltpu.VMEM((S, 128), jnp.float32),
        ],
    )
    def body(g4_hbm, logit_hbm, out_hbm, table_v, g_v, out_v):
        wid = lax.axis_index("s") * 2 + lax.axis_index("c")
        pltpu.sync_copy(logit_hbm, table_v)
        pltpu.sync_copy(g4_hbm.at[pl.ds(wid * DMAX, DMAX)], g_v)
        for b in range(B_PER_W):
            idx16 = g_v[b >> 2, pl.ds((b & 3) * DMAX, 16)]
            row = lax.shift_right_logical(idx16, 7)
            col = idx16 & 127
            val = plsc.load_gather(table_v, [row, col])
            out_v[b >> 3, pl.ds((b & 7) * 16, 16)] = val
        pltpu.sync_copy(out_v, out_hbm.at[pl.ds(wid * S, S)])

    return body


def _argmax_body(l2_ref, gum_ref, w_ref):
    lg = l2_ref[...]
    j2d = (128 * lax.broadcasted_iota(jnp.int32, (CROWS, 128), 0)
           + lax.broadcasted_iota(jnp.int32, (CROWS, 128), 1))
    io_c = lax.broadcasted_iota(jnp.int32, (8, 128), 1)
    wout = jnp.zeros((8, 128), jnp.int32)
    for k in range(S):
        val = lg + gum_ref[k]
        m = jnp.max(val)
        cand = jnp.where(val == m, j2d, jnp.int32(2 ** 30))
        wk = jnp.min(cand)
        wout = wout + jnp.where(io_c == k, wk, 0)
    w_ref[...] = wout


def _tc_argmax(l2, gum):
    return pl.pallas_call(
        _argmax_body,
        out_shape=jax.ShapeDtypeStruct((8, 128), jnp.int32),
    )(l2, gum)


def _select_body(w_ref, a_ref, g_ref, o1_ref, o2_ref):
    wrow = w_ref[...][0:1, :]
    io_c = lax.broadcasted_iota(jnp.int32, (128, 64), 0)
    io_col = lax.broadcasted_iota(jnp.int32, (128, 64), 1)
    mm = lax.shift_right_logical(io_col, 4) * DMAX
    kk = io_col & 15
    io_fc = lax.broadcasted_iota(jnp.int32, (AROWS, 64), 1) & 15
    oh = jnp.zeros((128, 64), jnp.float32)
    fill = jnp.zeros((AROWS, 64), jnp.int32)
    for k in range(S):
        wk = wrow[0, k]
        valid = wk < DMAX
        oh = oh + jnp.where((kk == k) & (io_c == mm + wk) & valid, 1.0, 0.0)
        fill = fill + jnp.where((io_fc == k) & (~valid), 1, 0)
    dn = (((1,), (0,)), ((), ()))
    int_min = jnp.int32(-(2 ** 31))
    o1 = lax.dot_general(a_ref[...].astype(jnp.float32), oh, dn,
                         preferred_element_type=jnp.float32)
    o2 = lax.dot_general(g_ref[...].astype(jnp.float32), oh, dn,
                         preferred_element_type=jnp.float32)
    o1_ref[...] = jnp.where(fill > 0, int_min, o1.astype(jnp.int32))
    o2_ref[...] = jnp.where(fill > 0, int_min, o2.astype(jnp.int32))


def _tc_select(w8, a4, g4):
    return pl.pallas_call(
        _select_body,
        out_shape=(
            jax.ShapeDtypeStruct((AROWS, 64), jnp.int32),
            jax.ShapeDtypeStruct((AROWS, 64), jnp.int32),
        ),
    )(w8, a4, g4)


_GUM = None


def _np_threefry2x32_bits(seed, n):
    def rotl(v, r):
        return ((v << np.uint32(r)) | (v >> np.uint32(32 - r))).astype(
            np.uint32)

    k0 = np.uint32(seed >> 32)
    k1 = np.uint32(seed & 0xFFFFFFFF)
    ks2 = np.uint32(k0 ^ k1 ^ np.uint32(0x1BD11BDA))
    rot = ((13, 15, 26, 6), (17, 29, 16, 24))
    sched = ((k1, ks2), (ks2, k0), (k0, k1), (k1, ks2), (ks2, k0))
    x0 = np.full(n, k0, dtype=np.uint32)
    x1 = (np.arange(n, dtype=np.uint32) + k1).astype(np.uint32)
    for i in range(5):
        for r in rot[i % 2]:
            x0 = (x0 + x1).astype(np.uint32)
            x1 = rotl(x1, r)
            x1 = x1 ^ x0
        x0 = (x0 + sched[i][0]).astype(np.uint32)
        x1 = (x1 + sched[i][1] + np.uint32(i + 1)).astype(np.uint32)
    return x0 ^ x1


def _gumbel_const():
    global _GUM
    if _GUM is None:
        bits = _np_threefry2x32_bits(42, DMAX * B * S)[: S * B * S]
        float_bits = (bits >> np.uint32(9)) | np.uint32(0x3F800000)
        f = float_bits.view(np.float32) - np.float32(1.0)
        tiny = np.float32(np.finfo(np.float32).tiny)
        u = np.maximum(tiny, (f * (np.float32(1.0) - tiny) + tiny).astype(
            np.float32))
        g = (-np.log(-np.log(u))).astype(np.float32)
        _GUM = g.reshape(S, CROWS, 128)
    return _GUM


def kernel(ids, geto_ids, geto_elms, adj_info, geto_adj_info, num_samples,
           support_size, batch_size, geto_dims, hop, resampling_rate):
    t8 = _sc_target_fn()(ids, geto_elms)
    logit_all = _tc_logits(t8, geto_elms)
    g4 = _sc_gather_one_fn()(geto_ids, geto_adj_info)
    l2 = _sc_gather_logits_fn()(g4, logit_all)
    gum = jnp.asarray(_gumbel_const())
    w8 = _tc_argmax(l2, gum)

    def _with_gather(operands):
        ids_, adj_, g4_, w8_ = operands
        a4 = _sc_gather_one_fn()(ids_, adj_)
        o1, o2 = _tc_select(w8_, a4, g4_)
        return o1.reshape(B, S), o2.reshape(B, S)

    def _all_fill(operands):
        del operands
        full = jnp.full((B, S), jnp.int32(-(2 ** 31)), jnp.int32)
        return full, full

    any_valid = jnp.any(w8[0, :S] < DMAX)
    return lax.cond(any_valid, _with_gather, _all_fill,
                    (ids, adj_info, g4, w8))
